# baseline (speedup 1.0000x reference)
import functools

import jax
import jax.numpy as jnp
from jax import lax
from jax.experimental import pallas as pl
from jax.experimental.pallas import tpu as pltpu
from jax.experimental.pallas import tpu_sc as plsc

N_NODES = 10000
N_EDGES = 320000
CH = 128
NPAD = 10240
NC, NS = 2, 16
EPT = N_EDGES // (NC * NS)
EPS = N_EDGES // NS
DEG_CHUNK = 2000
AGG_CHUNK = 128
EPT_PAD = 10240
ROWS_PER_TILE = NPAD // NS
OUT_CHUNK = 64

_mesh = plsc.VectorSubcoreMesh(core_axis_name="c", subcore_axis_name="s")


XROWS = NPAD // (NC * NS)
NHC = EPS // DEG_CHUNK


@functools.partial(
    pl.kernel,
    out_type=(jax.ShapeDtypeStruct((NPAD,), jnp.float32),
              jax.ShapeDtypeStruct((NPAD, CH), jnp.float32)),
    mesh=_mesh,
    scratch_types=[
        pltpu.VMEM((DEG_CHUNK,), jnp.int32),
        pltpu.VMEM((DEG_CHUNK,), jnp.int32),
        pltpu.VMEM((NPAD,), jnp.float32),
        pltpu.VMEM_SHARED((NS, NPAD), jnp.float32),
        pltpu.VMEM((ROWS_PER_TILE,), jnp.float32),
        pltpu.VMEM((ROWS_PER_TILE,), jnp.float32),
        pltpu.VMEM((ROWS_PER_TILE,), jnp.float32),
        pltpu.VMEM((XROWS, CH), jnp.float32),
        pltpu.SemaphoreType.DMA,
        pltpu.SemaphoreType.DMA,
        pltpu.SemaphoreType.DMA,
        pltpu.SemaphoreType.DMA,
        pltpu.SemaphoreType.DMA,
    ],
    compiler_params=pltpu.CompilerParams(needs_layout_passes=False),
)
def _dinv_kernel(dst_hbm, x_hbm, dinv_hbm, xs_hbm,
                 dst0, dst1, hist_v, hist_sh, acc_v, tmp0, tmp1, xrow_v,
                 hsem0, hsem1, rsem0, rsem1, xsem):
    c = lax.axis_index("c")
    s = lax.axis_index("s")
    zeros = jnp.zeros((16,), jnp.float32)
    ones = jnp.ones((16,), jnp.float32)
    base = s * ROWS_PER_TILE
    rbase = base + c * XROWS

    pltpu.async_copy(x_hbm.at[pl.ds(rbase, XROWS)], xrow_v, xsem)

    def _hstart(k, buf, sem):
        pltpu.async_copy(
            dst_hbm.at[pl.ds(s * EPS + k * DEG_CHUNK, DEG_CHUNK)], buf, sem)

    def _hwait(buf, sem):
        pltpu.make_async_copy(dst_hbm.at[pl.ds(0, DEG_CHUNK)], buf, sem).wait()

    _hstart(0, dst0, hsem0)
    _hstart(1, dst1, hsem1)

    def _zero_hist(i, carry):
        for u in range(8):
            hist_v[pl.ds((i * 8 + u) * 16, 16)] = zeros
        return carry

    lax.fori_loop(0, NPAD // 128, _zero_hist, 0)

    def _hbody(buf):
        def _grp(j, carry2):
            for u in range(5):
                idx = buf[pl.ds((j * 5 + u) * 16, 16)]
                plsc.addupdate_scatter(hist_v, [idx], ones)
            return carry2
        lax.fori_loop(0, DEG_CHUNK // 80, _grp, 0)

    def _hpair(h, carry):
        k = 2 * h
        _hwait(dst0, hsem0)
        _hbody(dst0)
        _hstart(k + 2, dst0, hsem0)
        _hwait(dst1, hsem1)
        _hbody(dst1)
        _hstart(k + 3, dst1, hsem1)
        return carry

    lax.fori_loop(0, NHC // 2 - 1, _hpair, 0)
    _hwait(dst0, hsem0)
    _hbody(dst0)
    _hwait(dst1, hsem1)
    _hbody(dst1)

    pltpu.sync_copy(hist_v, hist_sh.at[s])
    plsc.subcore_barrier()

    def _rstart(a, buf, sem):
        pltpu.async_copy(hist_sh.at[a, pl.ds(base, ROWS_PER_TILE)], buf, sem)

    def _rwait(buf, sem):
        pltpu.make_async_copy(
            hist_sh.at[0, pl.ds(base, ROWS_PER_TILE)], buf, sem).wait()

    _rstart(0, tmp0, rsem0)
    _rstart(1, tmp1, rsem1)

    def _zero_acc(i, carry):
        for u in range(8):
            acc_v[pl.ds((i * 8 + u) * 16, 16)] = zeros
        return carry

    lax.fori_loop(0, ROWS_PER_TILE // 128, _zero_acc, 0)

    def _radd(buf):
        def _addv(i, carry2):
            for u in range(5):
                o = (i * 5 + u) * 16
                acc_v[pl.ds(o, 16)] = acc_v[pl.ds(o, 16)] + buf[pl.ds(o, 16)]
            return carry2
        lax.fori_loop(0, ROWS_PER_TILE // 80, _addv, 0)

    def _rpair(h, carry):
        a = 2 * h
        _rwait(tmp0, rsem0)
        _radd(tmp0)
        _rstart(a + 2, tmp0, rsem0)
        _rwait(tmp1, rsem1)
        _radd(tmp1)
        _rstart(a + 3, tmp1, rsem1)
        return carry

    lax.fori_loop(0, NS // 2 - 1, _rpair, 0)
    _rwait(tmp0, rsem0)
    _radd(tmp0)
    _rwait(tmp1, rsem1)
    _radd(tmp1)

    def _newton(i, carry):
        d = acc_v[pl.ds(i * 16, 16)] + 1.0
        half = 0.5 * d
        y = plsc.bitcast(
            jnp.int32(0x5F3759DF) - (plsc.bitcast(d, jnp.int32) >> 1), jnp.float32)
        y = y * (1.5 - half * y * y)
        y = y * (1.5 - half * y * y)
        y = y * (1.5 - half * y * y)
        acc_v[pl.ds(i * 16, 16)] = y
        return carry

    lax.fori_loop(0, ROWS_PER_TILE // 16, _newton, 0)

    @pl.when(c == 0)
    def _write():
        pltpu.sync_copy(acc_v, dinv_hbm.at[pl.ds(base, ROWS_PER_TILE)])

    pltpu.make_async_copy(x_hbm.at[pl.ds(rbase, XROWS)], xrow_v, xsem).wait()

    def _scale(r, carry):
        dv = plsc.load_gather(
            acc_v, [jnp.full((16,), c * XROWS + r, dtype=jnp.int32)])
        for q in range(CH // 16):
            xrow_v[r, pl.ds(q * 16, 16)] = xrow_v[r, pl.ds(q * 16, 16)] * dv
        return carry

    lax.fori_loop(0, XROWS, _scale, 0)
    pltpu.sync_copy(xrow_v, xs_hbm.at[pl.ds(rbase, XROWS)])


NCHUNK = EPT_PAD // AGG_CHUNK
HALF = AGG_CHUNK // 2


@functools.partial(
    pl.kernel,
    out_type=jax.ShapeDtypeStruct((NC, NPAD, CH), jnp.float32),
    mesh=_mesh,
    scratch_types=[
        pltpu.VMEM((AGG_CHUNK,), jnp.int32),
        pltpu.VMEM((AGG_CHUNK,), jnp.int32),
        pltpu.VMEM((2, HALF), jnp.int32),
        pltpu.VMEM((2, HALF), jnp.int32),
        pltpu.VMEM((2, HALF), jnp.int32),
        pltpu.VMEM((2, HALF), jnp.int32),
        pltpu.VMEM((AGG_CHUNK, CH), jnp.float32),
        pltpu.VMEM((AGG_CHUNK, CH), jnp.float32),
        pltpu.VMEM_SHARED((NPAD, CH), jnp.float32),
        pltpu.VMEM((OUT_CHUNK, CH), jnp.float32),
        pltpu.SemaphoreType.DMA,
        pltpu.SemaphoreType.DMA,
        pltpu.SemaphoreType.DMA,
        pltpu.SemaphoreType.DMA,
        pltpu.SemaphoreType.DMA,
        pltpu.SemaphoreType.DMA,
    ],
    compiler_params=pltpu.CompilerParams(needs_layout_passes=False),
)
def _agg_kernel(src_hbm, dst_hbm, xs_hbm, out_hbm,
                si0, si1, d0, d1, d2, d3, r0, r1, acc_sh, buf_v,
                ia, ib, ga, gb, sa, sb):
    c = lax.axis_index("c")
    s = lax.axis_index("s")
    zeros = jnp.zeros((16,), jnp.float32)
    tid = c * NS + s
    tbase = tid * EPT_PAD
    si = [si0, si1]
    di = [d0, d1, d2, d3]
    rows = [r0, r1]
    isem = [ia, ib]
    gsem = [ga, gb]
    ssem = [sa, sb]

    def _start_i(g, sib, dib, sem):
        off = tbase + g * AGG_CHUNK
        pltpu.async_copy(src_hbm.at[pl.ds(off, AGG_CHUNK)], sib, sem)
        pltpu.async_copy(dst_hbm.at[pl.ds(off, HALF)], dib.at[0], sem)
        pltpu.async_copy(dst_hbm.at[pl.ds(off + HALF, HALF)], dib.at[1], sem)

    def _wait_i(sem):
        pltpu.make_async_copy(src_hbm.at[pl.ds(0, AGG_CHUNK)], si[0], sem).wait()
        pltpu.make_async_copy(dst_hbm.at[pl.ds(0, HALF)], di[0].at[0], sem).wait()
        pltpu.make_async_copy(dst_hbm.at[pl.ds(0, HALF)], di[0].at[1], sem).wait()

    def _start_g(sib, rb, sem):
        pltpu.async_copy(xs_hbm.at[sib.at[pl.ds(0, HALF)]],
                         rb.at[pl.ds(0, HALF)], sem)
        pltpu.async_copy(xs_hbm.at[sib.at[pl.ds(HALF, HALF)]],
                         rb.at[pl.ds(HALF, HALF)], sem)

    def _wait_g(rb, sem):
        for _ in range(2):
            pltpu.make_async_copy(
                xs_hbm.at[si0.at[pl.ds(0, HALF)]],
                rows[0].at[pl.ds(0, HALF)], sem).wait()

    def _start_s(dib, rb, sem):
        pltpu.async_copy(rb.at[pl.ds(0, HALF)], acc_sh.at[dib.at[0]], sem, add=True)
        pltpu.async_copy(rb.at[pl.ds(HALF, HALF)], acc_sh.at[dib.at[1]], sem, add=True)

    def _wait_s(sem):
        for _ in range(2):
            pltpu.make_async_copy(
                rows[0].at[pl.ds(0, HALF)], acc_sh.at[di[0].at[0]], sem).wait()

    def _slot(g, u, do_c=True, do_d=True, do_ef=True):
        p = u % 2
        _wait_g(rows[p], gsem[p])
        _start_s(di[u], rows[p], ssem[p])
        if do_c:
            _wait_s(ssem[1 - p])
        if do_d:
            _start_i(g + 2, si[p], di[(u + 2) % 4], isem[p])
        if do_ef:
            _wait_i(isem[1 - p])
            _start_g(si[1 - p], rows[1 - p], gsem[1 - p])

    _start_i(0, si[0], di[0], isem[0])
    _start_i(1, si[1], di[1], isem[1])

    def _zero_buf(r, carry):
        def _q(q, carry2):
            buf_v[r, pl.ds(q * 16, 16)] = zeros
            return carry2
        lax.fori_loop(0, CH // 16, _q, 0)
        return carry

    lax.fori_loop(0, OUT_CHUNK, _zero_buf, 0)

    base = s * ROWS_PER_TILE
    for k in range(ROWS_PER_TILE // OUT_CHUNK):
        pltpu.sync_copy(buf_v, acc_sh.at[pl.ds(base + k * OUT_CHUNK, OUT_CHUNK)])

    _wait_i(isem[0])
    _start_g(si[0], rows[0], gsem[0])
    plsc.subcore_barrier()

    _slot(0, 0, do_c=False)

    def _body(h, carry):
        g = 4 * h + 1
        _slot(g, 1)
        _slot(g + 1, 2)
        _slot(g + 2, 3)
        _slot(g + 3, 0)
        return carry

    lax.fori_loop(0, (NCHUNK - 4) // 4, _body, 0)
    _slot(NCHUNK - 3, (NCHUNK - 3) % 4)
    _slot(NCHUNK - 2, (NCHUNK - 2) % 4, do_d=False)
    _slot(NCHUNK - 1, (NCHUNK - 1) % 4, do_c=True, do_d=False, do_ef=False)
    _wait_s(ssem[(NCHUNK - 1) % 2])
    plsc.subcore_barrier()

    for k in range(ROWS_PER_TILE // OUT_CHUNK):
        pltpu.sync_copy(acc_sh.at[pl.ds(base + k * OUT_CHUNK, OUT_CHUNK)], buf_v)
        pltpu.sync_copy(buf_v, out_hbm.at[c, pl.ds(base + k * OUT_CHUNK, OUT_CHUNK)])


def _final_body(p_ref, xs_ref, dinv_ref, w_ref, b_ref, o_ref):
    t = (p_ref[0] + p_ref[1] + xs_ref[...]) * dinv_ref[...]
    acc = jnp.dot(t, w_ref[...], preferred_element_type=jnp.float32)
    o_ref[...] = jnp.maximum(acc + b_ref[...], 0.0)


_ROWS_BLK = 400
_GRID = N_NODES // _ROWS_BLK


def kernel(x, edge_index, W, b):
    src = edge_index[0].astype(jnp.int32)
    dst = edge_index[1].astype(jnp.int32)
    x_pad = jnp.zeros((NPAD, CH), jnp.float32).at[:N_NODES].set(x)

    dinv, xs = _dinv_kernel(dst, x_pad)
    dinv2d = dinv.reshape(NPAD, 1)

    n_fill = NC * NS * EPT_PAD - N_EDGES
    fill = N_NODES + (jnp.arange(n_fill, dtype=jnp.int32) % (NPAD - N_NODES))
    src_p = jnp.concatenate([src, fill])
    dst_p = jnp.concatenate([dst, fill])
    p = _agg_kernel(src_p, dst_p, xs)

    out = pl.pallas_call(
        _final_body,
        grid=(_GRID,),
        in_specs=[
            pl.BlockSpec((NC, _ROWS_BLK, CH), lambda i: (0, i, 0)),
            pl.BlockSpec((_ROWS_BLK, CH), lambda i: (i, 0)),
            pl.BlockSpec((_ROWS_BLK, 1), lambda i: (i, 0)),
            pl.BlockSpec((CH, CH), lambda i: (0, 0)),
            pl.BlockSpec((1, CH), lambda i: (0, 0)),
        ],
        out_specs=pl.BlockSpec((_ROWS_BLK, CH), lambda i: (i, 0)),
        out_shape=jax.ShapeDtypeStruct((N_NODES, CH), jnp.float32),
    )(p, xs, dinv2d, W, b.reshape(1, CH))

    return out


# speedup vs baseline: 1.0008x; 1.0008x over previous
import functools

import jax
import jax.numpy as jnp
from jax import lax
from jax.experimental import pallas as pl
from jax.experimental.pallas import tpu as pltpu
from jax.experimental.pallas import tpu_sc as plsc

N_NODES = 10000
N_EDGES = 320000
CH = 128
NPAD = 10240
NC, NS = 2, 16
EPT = N_EDGES // (NC * NS)
EPS = N_EDGES // NS
DEG_CHUNK = 2000
AGG_CHUNK = 128
EPT_PAD = 10240
ROWS_PER_TILE = NPAD // NS
OUT_CHUNK = 64

_mesh = plsc.VectorSubcoreMesh(core_axis_name="c", subcore_axis_name="s")


XROWS = NPAD // (NC * NS)
NHC = EPS // DEG_CHUNK


@functools.partial(
    pl.kernel,
    out_type=(jax.ShapeDtypeStruct((NPAD,), jnp.float32),
              jax.ShapeDtypeStruct((NPAD, CH), jnp.float32)),
    mesh=_mesh,
    scratch_types=[
        pltpu.VMEM((DEG_CHUNK,), jnp.int32),
        pltpu.VMEM((DEG_CHUNK,), jnp.int32),
        pltpu.VMEM((NPAD,), jnp.float32),
        pltpu.VMEM_SHARED((NS, NPAD), jnp.float32),
        pltpu.VMEM((ROWS_PER_TILE,), jnp.float32),
        pltpu.VMEM((ROWS_PER_TILE,), jnp.float32),
        pltpu.VMEM((ROWS_PER_TILE,), jnp.float32),
        pltpu.VMEM((XROWS, CH), jnp.float32),
        pltpu.SemaphoreType.DMA,
        pltpu.SemaphoreType.DMA,
        pltpu.SemaphoreType.DMA,
        pltpu.SemaphoreType.DMA,
        pltpu.SemaphoreType.DMA,
    ],
    compiler_params=pltpu.CompilerParams(needs_layout_passes=False),
)
def _dinv_kernel(dst_hbm, x_hbm, dinv_hbm, xs_hbm,
                 dst0, dst1, hist_v, hist_sh, acc_v, tmp0, tmp1, xrow_v,
                 hsem0, hsem1, rsem0, rsem1, xsem):
    c = lax.axis_index("c")
    s = lax.axis_index("s")
    zeros = jnp.zeros((16,), jnp.float32)
    ones = jnp.ones((16,), jnp.float32)
    base = s * ROWS_PER_TILE
    rbase = base + c * XROWS

    pltpu.async_copy(x_hbm.at[pl.ds(rbase, XROWS)], xrow_v, xsem)

    def _hstart(k, buf, sem):
        pltpu.async_copy(
            dst_hbm.at[pl.ds(s * EPS + k * DEG_CHUNK, DEG_CHUNK)], buf, sem)

    def _hwait(buf, sem):
        pltpu.make_async_copy(dst_hbm.at[pl.ds(0, DEG_CHUNK)], buf, sem).wait()

    _hstart(0, dst0, hsem0)
    _hstart(1, dst1, hsem1)

    def _zero_hist(i, carry):
        for u in range(8):
            hist_v[pl.ds((i * 8 + u) * 16, 16)] = zeros
        return carry

    lax.fori_loop(0, NPAD // 128, _zero_hist, 0)

    def _hbody(buf):
        def _grp(j, carry2):
            for u in range(5):
                idx = buf[pl.ds((j * 5 + u) * 16, 16)]
                plsc.addupdate_scatter(hist_v, [idx], ones)
            return carry2
        lax.fori_loop(0, DEG_CHUNK // 80, _grp, 0)

    def _hpair(h, carry):
        k = 2 * h
        _hwait(dst0, hsem0)
        _hbody(dst0)
        _hstart(k + 2, dst0, hsem0)
        _hwait(dst1, hsem1)
        _hbody(dst1)
        _hstart(k + 3, dst1, hsem1)
        return carry

    lax.fori_loop(0, NHC // 2 - 1, _hpair, 0)
    _hwait(dst0, hsem0)
    _hbody(dst0)
    _hwait(dst1, hsem1)
    _hbody(dst1)

    pltpu.sync_copy(hist_v, hist_sh.at[s])
    plsc.subcore_barrier()

    def _rstart(a, buf, sem):
        pltpu.async_copy(hist_sh.at[a, pl.ds(base, ROWS_PER_TILE)], buf, sem)

    def _rwait(buf, sem):
        pltpu.make_async_copy(
            hist_sh.at[0, pl.ds(base, ROWS_PER_TILE)], buf, sem).wait()

    _rstart(0, tmp0, rsem0)
    _rstart(1, tmp1, rsem1)

    def _zero_acc(i, carry):
        for u in range(8):
            acc_v[pl.ds((i * 8 + u) * 16, 16)] = zeros
        return carry

    lax.fori_loop(0, ROWS_PER_TILE // 128, _zero_acc, 0)

    def _radd(buf):
        def _addv(i, carry2):
            for u in range(5):
                o = (i * 5 + u) * 16
                acc_v[pl.ds(o, 16)] = acc_v[pl.ds(o, 16)] + buf[pl.ds(o, 16)]
            return carry2
        lax.fori_loop(0, ROWS_PER_TILE // 80, _addv, 0)

    def _rpair(h, carry):
        a = 2 * h
        _rwait(tmp0, rsem0)
        _radd(tmp0)
        _rstart(a + 2, tmp0, rsem0)
        _rwait(tmp1, rsem1)
        _radd(tmp1)
        _rstart(a + 3, tmp1, rsem1)
        return carry

    lax.fori_loop(0, NS // 2 - 1, _rpair, 0)
    _rwait(tmp0, rsem0)
    _radd(tmp0)
    _rwait(tmp1, rsem1)
    _radd(tmp1)

    def _newton(i, carry):
        d = acc_v[pl.ds(i * 16, 16)] + 1.0
        half = 0.5 * d
        y = plsc.bitcast(
            jnp.int32(0x5F3759DF) - (plsc.bitcast(d, jnp.int32) >> 1), jnp.float32)
        y = y * (1.5 - half * y * y)
        y = y * (1.5 - half * y * y)
        y = y * (1.5 - half * y * y)
        acc_v[pl.ds(i * 16, 16)] = y
        return carry

    lax.fori_loop(0, ROWS_PER_TILE // 16, _newton, 0)

    @pl.when(c == 0)
    def _write():
        pltpu.sync_copy(acc_v, dinv_hbm.at[pl.ds(base, ROWS_PER_TILE)])

    pltpu.make_async_copy(x_hbm.at[pl.ds(rbase, XROWS)], xrow_v, xsem).wait()

    def _scale(r, carry):
        dv = plsc.load_gather(
            acc_v, [jnp.full((16,), c * XROWS + r, dtype=jnp.int32)])
        for q in range(CH // 16):
            xrow_v[r, pl.ds(q * 16, 16)] = xrow_v[r, pl.ds(q * 16, 16)] * dv
        return carry

    lax.fori_loop(0, XROWS, _scale, 0)
    pltpu.sync_copy(xrow_v, xs_hbm.at[pl.ds(rbase, XROWS)])


NCHUNK = EPT_PAD // AGG_CHUNK
HALF = AGG_CHUNK // 2


@functools.partial(
    pl.kernel,
    out_type=jax.ShapeDtypeStruct((NC, NPAD, CH), jnp.float32),
    mesh=_mesh,
    scratch_types=[
        pltpu.VMEM((AGG_CHUNK,), jnp.int32),
        pltpu.VMEM((AGG_CHUNK,), jnp.int32),
        pltpu.VMEM((2, HALF), jnp.int32),
        pltpu.VMEM((2, HALF), jnp.int32),
        pltpu.VMEM((2, HALF), jnp.int32),
        pltpu.VMEM((2, HALF), jnp.int32),
        pltpu.VMEM((AGG_CHUNK, CH), jnp.float32),
        pltpu.VMEM((AGG_CHUNK, CH), jnp.float32),
        pltpu.VMEM_SHARED((NPAD, CH), jnp.float32),
        pltpu.VMEM((OUT_CHUNK, CH), jnp.float32),
        pltpu.SemaphoreType.DMA,
        pltpu.SemaphoreType.DMA,
        pltpu.SemaphoreType.DMA,
        pltpu.SemaphoreType.DMA,
        pltpu.SemaphoreType.DMA,
        pltpu.SemaphoreType.DMA,
    ],
    compiler_params=pltpu.CompilerParams(needs_layout_passes=False),
)
def _agg_kernel(src_hbm, dst_hbm, xs_hbm, out_hbm,
                si0, si1, d0, d1, d2, d3, r0, r1, acc_sh, buf_v,
                ia, ib, ga, gb, sa, sb):
    c = lax.axis_index("c")
    s = lax.axis_index("s")
    zeros = jnp.zeros((16,), jnp.float32)
    tid = c * NS + s
    tbase = tid * EPT_PAD
    si = [si0, si1]
    di = [d0, d1, d2, d3]
    rows = [r0, r1]
    isem = [ia, ib]
    gsem = [ga, gb]
    ssem = [sa, sb]

    def _start_i(g, sib, dib, sem):
        off = tbase + g * AGG_CHUNK
        pltpu.async_copy(src_hbm.at[pl.ds(off, AGG_CHUNK)], sib, sem)
        pltpu.async_copy(dst_hbm.at[pl.ds(off, HALF)], dib.at[0], sem)
        pltpu.async_copy(dst_hbm.at[pl.ds(off + HALF, HALF)], dib.at[1], sem)

    def _wait_i(sem):
        pltpu.make_async_copy(src_hbm.at[pl.ds(0, AGG_CHUNK)], si[0], sem).wait()
        pltpu.make_async_copy(dst_hbm.at[pl.ds(0, HALF)], di[0].at[0], sem).wait()
        pltpu.make_async_copy(dst_hbm.at[pl.ds(0, HALF)], di[0].at[1], sem).wait()

    def _start_g(sib, rb, sem):
        pltpu.async_copy(xs_hbm.at[sib], rb, sem)

    def _wait_g(rb, sem):
        pltpu.make_async_copy(xs_hbm.at[si0], rb, sem).wait()

    def _start_s(dib, rb, sem):
        pltpu.async_copy(rb.at[pl.ds(0, HALF)], acc_sh.at[dib.at[0]], sem, add=True)
        pltpu.async_copy(rb.at[pl.ds(HALF, HALF)], acc_sh.at[dib.at[1]], sem, add=True)

    def _wait_s(sem):
        for _ in range(2):
            pltpu.make_async_copy(
                rows[0].at[pl.ds(0, HALF)], acc_sh.at[di[0].at[0]], sem).wait()

    def _slot(g, u, do_c=True, do_d=True, do_ef=True):
        p = u % 2
        _wait_g(rows[p], gsem[p])
        _start_s(di[u], rows[p], ssem[p])
        if do_c:
            _wait_s(ssem[1 - p])
        if do_d:
            _start_i(g + 2, si[p], di[(u + 2) % 4], isem[p])
        if do_ef:
            _wait_i(isem[1 - p])
            _start_g(si[1 - p], rows[1 - p], gsem[1 - p])

    _start_i(0, si[0], di[0], isem[0])
    _start_i(1, si[1], di[1], isem[1])

    def _zero_buf(r, carry):
        def _q(q, carry2):
            buf_v[r, pl.ds(q * 16, 16)] = zeros
            return carry2
        lax.fori_loop(0, CH // 16, _q, 0)
        return carry

    lax.fori_loop(0, OUT_CHUNK, _zero_buf, 0)

    base = s * ROWS_PER_TILE
    for k in range(ROWS_PER_TILE // OUT_CHUNK):
        pltpu.sync_copy(buf_v, acc_sh.at[pl.ds(base + k * OUT_CHUNK, OUT_CHUNK)])

    _wait_i(isem[0])
    _start_g(si[0], rows[0], gsem[0])
    plsc.subcore_barrier()

    _slot(0, 0, do_c=False)

    def _body(h, carry):
        g = 4 * h + 1
        _slot(g, 1)
        _slot(g + 1, 2)
        _slot(g + 2, 3)
        _slot(g + 3, 0)
        return carry

    lax.fori_loop(0, (NCHUNK - 4) // 4, _body, 0)
    _slot(NCHUNK - 3, (NCHUNK - 3) % 4)
    _slot(NCHUNK - 2, (NCHUNK - 2) % 4, do_d=False)
    _slot(NCHUNK - 1, (NCHUNK - 1) % 4, do_c=True, do_d=False, do_ef=False)
    _wait_s(ssem[(NCHUNK - 1) % 2])
    plsc.subcore_barrier()

    for k in range(ROWS_PER_TILE // OUT_CHUNK):
        pltpu.sync_copy(acc_sh.at[pl.ds(base + k * OUT_CHUNK, OUT_CHUNK)], buf_v)
        pltpu.sync_copy(buf_v, out_hbm.at[c, pl.ds(base + k * OUT_CHUNK, OUT_CHUNK)])


def _final_body(p_ref, xs_ref, dinv_ref, w_ref, b_ref, o_ref):
    t = (p_ref[0] + p_ref[1] + xs_ref[...]) * dinv_ref[...]
    acc = jnp.dot(t, w_ref[...], preferred_element_type=jnp.float32)
    o_ref[...] = jnp.maximum(acc + b_ref[...], 0.0)


_ROWS_BLK = 400
_GRID = N_NODES // _ROWS_BLK


def kernel(x, edge_index, W, b):
    src = edge_index[0].astype(jnp.int32)
    dst = edge_index[1].astype(jnp.int32)
    x_pad = jnp.zeros((NPAD, CH), jnp.float32).at[:N_NODES].set(x)

    dinv, xs = _dinv_kernel(dst, x_pad)
    dinv2d = dinv.reshape(NPAD, 1)

    n_fill = NC * NS * EPT_PAD - N_EDGES
    fill = N_NODES + (jnp.arange(n_fill, dtype=jnp.int32) % (NPAD - N_NODES))
    src_p = jnp.concatenate([src, fill])
    dst_p = jnp.concatenate([dst, fill])
    p = _agg_kernel(src_p, dst_p, xs)

    out = pl.pallas_call(
        _final_body,
        grid=(_GRID,),
        in_specs=[
            pl.BlockSpec((NC, _ROWS_BLK, CH), lambda i: (0, i, 0)),
            pl.BlockSpec((_ROWS_BLK, CH), lambda i: (i, 0)),
            pl.BlockSpec((_ROWS_BLK, 1), lambda i: (i, 0)),
            pl.BlockSpec((CH, CH), lambda i: (0, 0)),
            pl.BlockSpec((1, CH), lambda i: (0, 0)),
        ],
        out_specs=pl.BlockSpec((_ROWS_BLK, CH), lambda i: (i, 0)),
        out_shape=jax.ShapeDtypeStruct((N_NODES, CH), jnp.float32),
    )(p, xs, dinv2d, W, b.reshape(1, CH))

    return out
